# Initial kernel scaffold; baseline (speedup 1.0000x reference)
import jax
import jax.numpy as jnp
from jax.experimental import pallas as pl


def kernel(query_0, query_1, reference_points, input_flatten_0, input_flatten_1, input_spatial_shapes, input_level_start_index, W_value, b_value, W_off, b_off, W_attn, b_attn, W_out, b_out, W_agg, b_agg):
    raise NotImplementedError("write your pallas kernel here")


# speedup vs baseline: 9.6516x; 9.6516x over previous
import functools

import jax
import jax.numpy as jnp
from jax import lax
from jax.experimental import pallas as pl
from jax.experimental.pallas import tpu as pltpu
from jax.experimental.pallas import tpu_sc as plsc

F32 = jnp.float32
I32 = jnp.int32

D = 256
H = 8
L = 4
P = 4
HD = 32
NF = 2
NB = 2
LQ = 1024
LEVELS = [(64, 48), (32, 96), (16, 16), (16, 16)]
LEN_IN = sum(h * w for h, w in LEVELS)
STARTS = [0, 3072, 6144, 6400]
NFB = NF * NB
ROWS = NFB * LQ * H
TABLE_ROWS = NFB * LEN_IN * H
K = L * P * 4

NW = 32
RPW = ROWS // NW
CHUNK = 16
NCHUNK = RPW // CHUNK
IDX_PER_CHUNK = CHUNK * K


def _mm_bias_body(x_ref, w_ref, b_ref, o_ref):
    acc = jnp.dot(x_ref[0], w_ref[...], preferred_element_type=F32)
    o_ref[...] = (acc + b_ref[...])[None]


def _matmul_bias(x, wt, b2, row_tile):
    g0, rows, _ = x.shape
    dout = wt.shape[1]
    grid = (g0, rows // row_tile)
    return pl.pallas_call(
        _mm_bias_body,
        grid=grid,
        in_specs=[
            pl.BlockSpec((1, row_tile, x.shape[2]), lambda i, j: (i, j, 0)),
            pl.BlockSpec((x.shape[2], dout), lambda i, j: (0, 0)),
            pl.BlockSpec((1, dout), lambda i, j: (0, 0)),
        ],
        out_specs=pl.BlockSpec((1, row_tile, dout), lambda i, j: (i, j, 0)),
        out_shape=jax.ShapeDtypeStruct((g0, rows, dout), F32),
    )(x, wt, b2)


def _softmax_body(x_ref, o_ref):
    x = x_ref[...]
    m = jnp.max(x, axis=-1, keepdims=True)
    e = jnp.exp(x - m)
    o_ref[...] = e / jnp.sum(e, axis=-1, keepdims=True)


def _softmax32(a):
    return pl.pallas_call(
        _softmax_body,
        grid=(NB, 4),
        in_specs=[pl.BlockSpec((1, LQ // 4, H, 32), lambda i, j: (i, j, 0, 0))],
        out_specs=pl.BlockSpec((1, LQ // 4, H, 32), lambda i, j: (i, j, 0, 0)),
        out_shape=jax.ShapeDtypeStruct(a.shape, F32),
    )(a)


def _idxw_body(ox_ref, oy_ref, rx_ref, ry_ref, aw_ref,
               i0, i1, i2, i3, w0, w1, w2, w3):
    fb = pl.program_id(0)
    lane = lax.broadcasted_iota(I32, (1, LQ, H * L * P), 2)
    h_lane = lane // (L * P)
    l_lane = (lane % (L * P)) // P
    wl = jnp.where(l_lane == 0, 48.0, jnp.where(l_lane == 1, 96.0, 16.0))
    hl = jnp.where(l_lane == 0, 64.0, jnp.where(l_lane == 1, 32.0, 16.0))
    start = jnp.where(l_lane == 0, 0,
                      jnp.where(l_lane == 1, 3072,
                                jnp.where(l_lane == 2, 6144, 6400)))
    x = rx_ref[...] * wl + ox_ref[...] - 0.5
    y = ry_ref[...] * hl + oy_ref[...] - 0.5
    x0 = jnp.floor(x)
    y0 = jnp.floor(y)
    wx1 = x - x0
    wx0 = 1.0 - wx1
    wy1 = y - y0
    wy0 = 1.0 - wy1
    aw = aw_ref[...]
    wi = wl.astype(I32)
    base = (fb * LEN_IN + start) * H + h_lane

    def corner(dx, dy, wb, iref, wref):
        xi = x0 + dx
        yi = y0 + dy
        valid = (xi >= 0.0) & (xi <= wl - 1.0) & (yi >= 0.0) & (yi <= hl - 1.0)
        xc = jnp.clip(xi, 0.0, wl - 1.0).astype(I32)
        yc = jnp.clip(yi, 0.0, hl - 1.0).astype(I32)
        iref[...] = base + (yc * wi + xc) * H
        wref[...] = wb * aw * valid.astype(F32)

    corner(0.0, 0.0, wx0 * wy0, i0, w0)
    corner(1.0, 0.0, wx1 * wy0, i1, w1)
    corner(0.0, 1.0, wx0 * wy1, i2, w2)
    corner(1.0, 1.0, wx1 * wy1, i3, w3)


def _idx_weights(ox, oy, rx, ry, aw):
    spec = pl.BlockSpec((1, LQ, H * L * P), lambda i: (i, 0, 0))
    shp = jax.ShapeDtypeStruct((NFB, LQ, H * L * P), F32)
    shpi = jax.ShapeDtypeStruct((NFB, LQ, H * L * P), I32)
    return pl.pallas_call(
        _idxw_body,
        grid=(NFB,),
        in_specs=[spec] * 5,
        out_specs=[spec] * 8,
        out_shape=[shpi] * 4 + [shp] * 4,
    )(ox, oy, rx, ry, aw)


def _final_body(s0_ref, s1_ref, wo_ref, bo_ref, g0_ref, g1_ref, bg_ref, o_ref):
    t0 = jnp.dot(s0_ref[0], wo_ref[...], preferred_element_type=F32) + bo_ref[...]
    t1 = jnp.dot(s1_ref[0], wo_ref[...], preferred_element_type=F32) + bo_ref[...]
    o = (jnp.dot(t0, g0_ref[...], preferred_element_type=F32)
         + jnp.dot(t1, g1_ref[...], preferred_element_type=F32) + bg_ref[...])
    o_ref[...] = o[None]


def _final_combine(s0, s1, wot, bo2, g0t, g1t, bg2):
    mspec = lambda shape: pl.BlockSpec(shape, lambda i: (0, 0))
    return pl.pallas_call(
        _final_body,
        grid=(NB,),
        in_specs=[
            pl.BlockSpec((1, LQ, D), lambda i: (i, 0, 0)),
            pl.BlockSpec((1, LQ, D), lambda i: (i, 0, 0)),
            mspec((D, D)), mspec((1, D)), mspec((D, D)), mspec((D, D)),
            mspec((1, D)),
        ],
        out_specs=pl.BlockSpec((1, LQ, D), lambda i: (i, 0, 0)),
        out_shape=jax.ShapeDtypeStruct((NB, LQ, D), F32),
    )(s0, s1, wot, bo2, g0t, g1t, bg2)


def _make_sc_combine():
    mesh = plsc.VectorSubcoreMesh(core_axis_name="c", subcore_axis_name="s")

    @functools.partial(
        pl.kernel,
        mesh=mesh,
        out_type=jax.ShapeDtypeStruct((ROWS, HD), F32),
        scratch_types=[
            pltpu.VMEM((8, 128), I32),
            pltpu.VMEM((IDX_PER_CHUNK,), F32),
            pltpu.VMEM((IDX_PER_CHUNK, HD), F32),
            pltpu.VMEM((CHUNK, HD), F32),
            pltpu.SemaphoreType.DMA,
        ],
        compiler_params=pltpu.CompilerParams(use_tc_tiling_on_sc=False),
    )
    def sc_combine(table, idxs, ws, out, idx_v, w_v, rows_v, out_v, sem):
        wid = lax.axis_index("s") * 2 + lax.axis_index("c")

        def chunk_body(c, carry):
            pltpu.sync_copy(idxs.at[wid, c], idx_v)
            pltpu.sync_copy(ws.at[wid, c], w_v)
            copies = [
                pltpu.async_copy(
                    table.at[idx_v.at[j]],
                    rows_v.at[pl.ds(j * 128, 128)], sem)
                for j in range(8)
            ]
            for cp in copies:
                cp.wait()
            def row_body(r, carry2):
                base = r * K
                a0 = jnp.zeros((16,), F32)
                a1 = jnp.zeros((16,), F32)
                for g in range(K // 16):
                    wvec = w_v[pl.ds(base + g * 16, 16)]
                    for j in range(16):
                        kk = base + g * 16 + j
                        ws_j = wvec[j]
                        a0 = a0 + ws_j * rows_v[kk, pl.ds(0, 16)]
                        a1 = a1 + ws_j * rows_v[kk, pl.ds(16, 16)]
                out_v[r, pl.ds(0, 16)] = a0
                out_v[r, pl.ds(16, 16)] = a1
                return carry2

            lax.fori_loop(0, CHUNK, row_body, 0)
            pltpu.sync_copy(out_v, out.at[pl.ds(wid * RPW + c * CHUNK, CHUNK)])
            return carry

        lax.fori_loop(0, NCHUNK, chunk_body, 0)

    return sc_combine


_sc_cache = []


def _sc_combine(table, idxs, ws):
    if not _sc_cache:
        _sc_cache.append(_make_sc_combine())
    return _sc_cache[0](table, idxs, ws)


@jax.jit
def kernel(query_0, query_1, reference_points, input_flatten_0, input_flatten_1,
           input_spatial_shapes, input_level_start_index, W_value, b_value,
           W_off, b_off, W_attn, b_attn, W_out, b_out, W_agg, b_agg):
    flat = jnp.stack([input_flatten_0, input_flatten_1]).reshape(NFB, LEN_IN, D)
    vals = _matmul_bias(flat, W_value.T, b_value[None], 512)
    table = vals.reshape(TABLE_ROWS, HD)

    q = jnp.stack([query_0, query_1]).reshape(NFB, LQ, D)
    wcat = jnp.concatenate([W_off.T, W_attn.T], axis=1)
    bcat = jnp.concatenate([b_off, b_attn])[None]
    logits = _matmul_bias(q, wcat, bcat, LQ)

    off = logits[:, :, :H * L * P * 2].reshape(NFB, LQ, H * L * P, 2)
    ox = off[..., 0]
    oy = off[..., 1]

    al = logits[:, :, H * L * P * 2:].reshape(NF, NB, LQ, H, L * P)
    acat = al.transpose(1, 2, 3, 0, 4).reshape(NB, LQ, H, NF * L * P)
    asm = _softmax32(acat)
    aw = (asm.reshape(NB, LQ, H, NF, L * P).transpose(3, 0, 1, 2, 4)
          .reshape(NFB, LQ, H * L * P))

    rx = jnp.broadcast_to(reference_points[:, :, None, :, None, 0],
                          (NB, LQ, H, L, P)).reshape(NB, LQ, H * L * P)
    ry = jnp.broadcast_to(reference_points[:, :, None, :, None, 1],
                          (NB, LQ, H, L, P)).reshape(NB, LQ, H * L * P)
    rx = jnp.broadcast_to(rx[None], (NF, NB, LQ, H * L * P)).reshape(NFB, LQ, -1)
    ry = jnp.broadcast_to(ry[None], (NF, NB, LQ, H * L * P)).reshape(NFB, LQ, -1)

    i0, i1, i2, i3, w0, w1, w2, w3 = _idx_weights(ox, oy, rx, ry, aw)

    def cols(parts, dtype):
        s = jnp.stack([p.reshape(NFB, LQ, H, L * P) for p in parts], axis=-1)
        return s.reshape(ROWS, K).astype(dtype)

    idx = cols([i0, i1, i2, i3], I32).reshape(NW, NCHUNK, 8, 128)
    wgt = cols([w0, w1, w2, w3], F32).reshape(NW, NCHUNK, IDX_PER_CHUNK)

    sampled = _sc_combine(table, idx, wgt)
    s = sampled.reshape(NF, NB, LQ, D)

    return _final_combine(
        s[0], s[1], W_out.T, b_out[None],
        W_agg[:, :D].T, W_agg[:, D:].T, b_agg[None])


# speedup vs baseline: 24.7520x; 2.5645x over previous
import functools

import jax
import jax.numpy as jnp
from jax import lax
from jax.experimental import pallas as pl
from jax.experimental.pallas import tpu as pltpu
from jax.experimental.pallas import tpu_sc as plsc

F32 = jnp.float32
I32 = jnp.int32

D = 256
H = 8
L = 4
P = 4
HD = 32
NF = 2
NB = 2
LQ = 1024
LEVELS = [(64, 48), (32, 96), (16, 16), (16, 16)]
LEN_IN = sum(h * w for h, w in LEVELS)
STARTS = [0, 3072, 6144, 6400]
NFB = NF * NB
ROWS = NFB * LQ * H
TABLE_ROWS = NFB * LEN_IN * H
K = L * P * 4

NW = 32
RPW = ROWS // NW
CHUNK = 16
NCHUNK = RPW // CHUNK
IDX_PER_CHUNK = CHUNK * K


def _mm_bias_body(x_ref, w_ref, b_ref, o_ref):
    acc = jnp.dot(x_ref[0], w_ref[...], preferred_element_type=F32)
    o_ref[...] = (acc + b_ref[...])[None]


def _matmul_bias(x, wt, b2, row_tile):
    g0, rows, _ = x.shape
    dout = wt.shape[1]
    grid = (g0, rows // row_tile)
    return pl.pallas_call(
        _mm_bias_body,
        grid=grid,
        in_specs=[
            pl.BlockSpec((1, row_tile, x.shape[2]), lambda i, j: (i, j, 0)),
            pl.BlockSpec((x.shape[2], dout), lambda i, j: (0, 0)),
            pl.BlockSpec((1, dout), lambda i, j: (0, 0)),
        ],
        out_specs=pl.BlockSpec((1, row_tile, dout), lambda i, j: (i, j, 0)),
        out_shape=jax.ShapeDtypeStruct((g0, rows, dout), F32),
    )(x, wt, b2)


def _softmax_body(x_ref, o_ref):
    x = x_ref[...]
    m = jnp.max(x, axis=-1, keepdims=True)
    e = jnp.exp(x - m)
    o_ref[...] = e / jnp.sum(e, axis=-1, keepdims=True)


def _softmax32(a):
    return pl.pallas_call(
        _softmax_body,
        grid=(NB, 4),
        in_specs=[pl.BlockSpec((1, LQ // 4, H, 32), lambda i, j: (i, j, 0, 0))],
        out_specs=pl.BlockSpec((1, LQ // 4, H, 32), lambda i, j: (i, j, 0, 0)),
        out_shape=jax.ShapeDtypeStruct(a.shape, F32),
    )(a)


def _idxw_body(ox_ref, oy_ref, rx_ref, ry_ref, aw_ref,
               i0, i1, i2, i3, w0, w1, w2, w3):
    fb = pl.program_id(0)
    lane = lax.broadcasted_iota(I32, (1, LQ, H * L * P), 2)
    h_lane = lane // (L * P)
    l_lane = (lane % (L * P)) // P
    wl = jnp.where(l_lane == 0, 48.0, jnp.where(l_lane == 1, 96.0, 16.0))
    hl = jnp.where(l_lane == 0, 64.0, jnp.where(l_lane == 1, 32.0, 16.0))
    start = jnp.where(l_lane == 0, 0,
                      jnp.where(l_lane == 1, 3072,
                                jnp.where(l_lane == 2, 6144, 6400)))
    x = rx_ref[...] * wl + ox_ref[...] - 0.5
    y = ry_ref[...] * hl + oy_ref[...] - 0.5
    x0 = jnp.floor(x)
    y0 = jnp.floor(y)
    wx1 = x - x0
    wx0 = 1.0 - wx1
    wy1 = y - y0
    wy0 = 1.0 - wy1
    aw = aw_ref[...]
    wi = wl.astype(I32)
    base = (fb * LEN_IN + start) * H + h_lane

    def corner(dx, dy, wb, iref, wref):
        xi = x0 + dx
        yi = y0 + dy
        valid = (xi >= 0.0) & (xi <= wl - 1.0) & (yi >= 0.0) & (yi <= hl - 1.0)
        xc = jnp.clip(xi, 0.0, wl - 1.0).astype(I32)
        yc = jnp.clip(yi, 0.0, hl - 1.0).astype(I32)
        iref[...] = base + (yc * wi + xc) * H
        wref[...] = wb * aw * valid.astype(F32)

    corner(0.0, 0.0, wx0 * wy0, i0, w0)
    corner(1.0, 0.0, wx1 * wy0, i1, w1)
    corner(0.0, 1.0, wx0 * wy1, i2, w2)
    corner(1.0, 1.0, wx1 * wy1, i3, w3)


def _idx_weights(ox, oy, rx, ry, aw):
    spec = pl.BlockSpec((1, LQ, H * L * P), lambda i: (i, 0, 0))
    shp = jax.ShapeDtypeStruct((NFB, LQ, H * L * P), F32)
    shpi = jax.ShapeDtypeStruct((NFB, LQ, H * L * P), I32)
    return pl.pallas_call(
        _idxw_body,
        grid=(NFB,),
        in_specs=[spec] * 5,
        out_specs=[spec] * 8,
        out_shape=[shpi] * 4 + [shp] * 4,
    )(ox, oy, rx, ry, aw)


def _final_body(s0_ref, s1_ref, wo_ref, bo_ref, g0_ref, g1_ref, bg_ref, o_ref):
    t0 = jnp.dot(s0_ref[0], wo_ref[...], preferred_element_type=F32) + bo_ref[...]
    t1 = jnp.dot(s1_ref[0], wo_ref[...], preferred_element_type=F32) + bo_ref[...]
    o = (jnp.dot(t0, g0_ref[...], preferred_element_type=F32)
         + jnp.dot(t1, g1_ref[...], preferred_element_type=F32) + bg_ref[...])
    o_ref[...] = o[None]


def _final_combine(s0, s1, wot, bo2, g0t, g1t, bg2):
    mspec = lambda shape: pl.BlockSpec(shape, lambda i: (0, 0))
    return pl.pallas_call(
        _final_body,
        grid=(NB,),
        in_specs=[
            pl.BlockSpec((1, LQ, D), lambda i: (i, 0, 0)),
            pl.BlockSpec((1, LQ, D), lambda i: (i, 0, 0)),
            mspec((D, D)), mspec((1, D)), mspec((D, D)), mspec((D, D)),
            mspec((1, D)),
        ],
        out_specs=pl.BlockSpec((1, LQ, D), lambda i: (i, 0, 0)),
        out_shape=jax.ShapeDtypeStruct((NB, LQ, D), F32),
    )(s0, s1, wot, bo2, g0t, g1t, bg2)


def _make_sc_combine():
    mesh = plsc.VectorSubcoreMesh(core_axis_name="c", subcore_axis_name="s")

    @functools.partial(
        pl.kernel,
        mesh=mesh,
        out_type=jax.ShapeDtypeStruct((ROWS, HD), F32),
        scratch_types=[
            pltpu.VMEM((8, 128), I32),
            pltpu.VMEM((8, 128), F32),
            pltpu.VMEM((IDX_PER_CHUNK, HD), F32),
            pltpu.VMEM((CHUNK, HD), F32),
            pltpu.SemaphoreType.DMA,
        ],
        compiler_params=pltpu.CompilerParams(use_tc_tiling_on_sc=False),
    )
    def sc_combine(table, idxs, ws, out, idx_v, w_v, rows_v, out_v, sem):
        wid = lax.axis_index("s") * 2 + lax.axis_index("c")

        def chunk_body(c, carry):
            row0 = wid * 128 + c * 2
            for cr in range(4):
                pltpu.sync_copy(idxs.at[cr, pl.ds(row0, 2)],
                                idx_v.at[pl.ds(cr * 2, 2)])
                pltpu.sync_copy(ws.at[cr, pl.ds(row0, 2)],
                                w_v.at[pl.ds(cr * 2, 2)])
            copies = [
                pltpu.async_copy(
                    table.at[idx_v.at[j]],
                    rows_v.at[pl.ds(j * 128, 128)], sem)
                for j in range(8)
            ]
            for cp in copies:
                cp.wait()
            def row_body(r, carry2):
                qq = r // 8
                h = r % 8
                a0 = jnp.zeros((16,), F32)
                a1 = jnp.zeros((16,), F32)
                for cr in range(4):
                    eb = (cr * 2 + qq) * 128 + h * 16
                    wvec = w_v[cr * 2 + qq, pl.ds(h * 16, 16)]
                    for j in range(16):
                        kk = eb + j
                        ws_j = wvec[j]
                        a0 = a0 + ws_j * rows_v[kk, pl.ds(0, 16)]
                        a1 = a1 + ws_j * rows_v[kk, pl.ds(16, 16)]
                out_v[r, pl.ds(0, 16)] = a0
                out_v[r, pl.ds(16, 16)] = a1
                return carry2

            lax.fori_loop(0, CHUNK, row_body, 0)
            pltpu.sync_copy(out_v, out.at[pl.ds(wid * RPW + c * CHUNK, CHUNK)])
            return carry

        lax.fori_loop(0, NCHUNK, chunk_body, 0)

    return sc_combine


_sc_cache = []


def _sc_combine(table, idxs, ws):
    if not _sc_cache:
        _sc_cache.append(_make_sc_combine())
    return _sc_cache[0](table, idxs, ws)


@jax.jit
def kernel(query_0, query_1, reference_points, input_flatten_0, input_flatten_1,
           input_spatial_shapes, input_level_start_index, W_value, b_value,
           W_off, b_off, W_attn, b_attn, W_out, b_out, W_agg, b_agg):
    flat = jnp.stack([input_flatten_0, input_flatten_1]).reshape(NFB, LEN_IN, D)
    vals = _matmul_bias(flat, W_value.T, b_value[None], 512)
    table = vals.reshape(TABLE_ROWS, HD)

    q = jnp.stack([query_0, query_1]).reshape(NFB, LQ, D)
    wcat = jnp.concatenate([W_off.T, W_attn.T], axis=1)
    bcat = jnp.concatenate([b_off, b_attn])[None]
    logits = _matmul_bias(q, wcat, bcat, LQ)

    off = logits[:, :, :H * L * P * 2].reshape(NFB, LQ, H * L * P, 2)
    ox = off[..., 0]
    oy = off[..., 1]

    al = logits[:, :, H * L * P * 2:].reshape(NF, NB, LQ, H, L * P)
    acat = al.transpose(1, 2, 3, 0, 4).reshape(NB, LQ, H, NF * L * P)
    asm = _softmax32(acat)
    aw = (asm.reshape(NB, LQ, H, NF, L * P).transpose(3, 0, 1, 2, 4)
          .reshape(NFB, LQ, H * L * P))

    rx = jnp.broadcast_to(reference_points[:, :, None, :, None, 0],
                          (NB, LQ, H, L, P)).reshape(NB, LQ, H * L * P)
    ry = jnp.broadcast_to(reference_points[:, :, None, :, None, 1],
                          (NB, LQ, H, L, P)).reshape(NB, LQ, H * L * P)
    rx = jnp.broadcast_to(rx[None], (NF, NB, LQ, H * L * P)).reshape(NFB, LQ, -1)
    ry = jnp.broadcast_to(ry[None], (NF, NB, LQ, H * L * P)).reshape(NFB, LQ, -1)

    i0, i1, i2, i3, w0, w1, w2, w3 = _idx_weights(ox, oy, rx, ry, aw)

    idx = jnp.stack([i0, i1, i2, i3]).reshape(4, NFB * LQ, H * L * P)
    wgt = jnp.stack([w0, w1, w2, w3]).reshape(4, NFB * LQ, H * L * P)

    sampled = _sc_combine(table, idx, wgt)
    s = sampled.reshape(NF, NB, LQ, D)

    return _final_combine(
        s[0], s[1], W_out.T, b_out[None],
        W_agg[:, :D].T, W_agg[:, D:].T, b_agg[None])


# speedup vs baseline: 30.3191x; 1.2249x over previous
import functools

import jax
import jax.numpy as jnp
from jax import lax
from jax.experimental import pallas as pl
from jax.experimental.pallas import tpu as pltpu
from jax.experimental.pallas import tpu_sc as plsc

F32 = jnp.float32
I32 = jnp.int32

D = 256
H = 8
L = 4
P = 4
HD = 32
NF = 2
NB = 2
LQ = 1024
LEVELS = [(64, 48), (32, 96), (16, 16), (16, 16)]
LEN_IN = sum(h * w for h, w in LEVELS)
STARTS = [0, 3072, 6144, 6400]
NFB = NF * NB
ROWS = NFB * LQ * H
TABLE_ROWS = NFB * LEN_IN * H
K = L * P * 4

NW = 32
RPW = ROWS // NW
CHUNK = 16
NCHUNK = RPW // CHUNK
IDX_PER_CHUNK = CHUNK * K


def _mm_bias_body(x_ref, w_ref, b_ref, o_ref):
    acc = jnp.dot(x_ref[0], w_ref[...], preferred_element_type=F32)
    o_ref[...] = (acc + b_ref[...])[None]


def _matmul_bias(x, wt, b2, row_tile):
    g0, rows, _ = x.shape
    dout = wt.shape[1]
    grid = (g0, rows // row_tile)
    return pl.pallas_call(
        _mm_bias_body,
        grid=grid,
        in_specs=[
            pl.BlockSpec((1, row_tile, x.shape[2]), lambda i, j: (i, j, 0)),
            pl.BlockSpec((x.shape[2], dout), lambda i, j: (0, 0)),
            pl.BlockSpec((1, dout), lambda i, j: (0, 0)),
        ],
        out_specs=pl.BlockSpec((1, row_tile, dout), lambda i, j: (i, j, 0)),
        out_shape=jax.ShapeDtypeStruct((g0, rows, dout), F32),
    )(x, wt, b2)


def _softmax_body(x_ref, o_ref):
    x = x_ref[...]
    m = jnp.max(x, axis=-1, keepdims=True)
    e = jnp.exp(x - m)
    o_ref[...] = e / jnp.sum(e, axis=-1, keepdims=True)


def _softmax32(a):
    return pl.pallas_call(
        _softmax_body,
        grid=(NB, 4),
        in_specs=[pl.BlockSpec((1, LQ // 4, H, 32), lambda i, j: (i, j, 0, 0))],
        out_specs=pl.BlockSpec((1, LQ // 4, H, 32), lambda i, j: (i, j, 0, 0)),
        out_shape=jax.ShapeDtypeStruct(a.shape, F32),
    )(a)


def _idxw_body(ox_ref, oy_ref, rx_ref, ry_ref, aw_ref,
               i0, i1, i2, i3, w0, w1, w2, w3):
    fb = pl.program_id(0)
    lane = lax.broadcasted_iota(I32, (1, LQ, H * L * P), 2)
    h_lane = lane // (L * P)
    l_lane = (lane % (L * P)) // P
    wl = jnp.where(l_lane == 0, 48.0, jnp.where(l_lane == 1, 96.0, 16.0))
    hl = jnp.where(l_lane == 0, 64.0, jnp.where(l_lane == 1, 32.0, 16.0))
    start = jnp.where(l_lane == 0, 0,
                      jnp.where(l_lane == 1, 3072,
                                jnp.where(l_lane == 2, 6144, 6400)))
    x = rx_ref[...] * wl + ox_ref[...] - 0.5
    y = ry_ref[...] * hl + oy_ref[...] - 0.5
    x0 = jnp.floor(x)
    y0 = jnp.floor(y)
    wx1 = x - x0
    wx0 = 1.0 - wx1
    wy1 = y - y0
    wy0 = 1.0 - wy1
    aw = aw_ref[...]
    wi = wl.astype(I32)
    base = (fb * LEN_IN + start) * H + h_lane

    def corner(dx, dy, wb, iref, wref):
        xi = x0 + dx
        yi = y0 + dy
        valid = (xi >= 0.0) & (xi <= wl - 1.0) & (yi >= 0.0) & (yi <= hl - 1.0)
        xc = jnp.clip(xi, 0.0, wl - 1.0).astype(I32)
        yc = jnp.clip(yi, 0.0, hl - 1.0).astype(I32)
        iref[...] = base + (yc * wi + xc) * H
        wref[...] = wb * aw * valid.astype(F32)

    corner(0.0, 0.0, wx0 * wy0, i0, w0)
    corner(1.0, 0.0, wx1 * wy0, i1, w1)
    corner(0.0, 1.0, wx0 * wy1, i2, w2)
    corner(1.0, 1.0, wx1 * wy1, i3, w3)


def _idx_weights(ox, oy, rx, ry, aw):
    spec = pl.BlockSpec((1, LQ, H * L * P), lambda i: (i, 0, 0))
    shp = jax.ShapeDtypeStruct((NFB, LQ, H * L * P), F32)
    shpi = jax.ShapeDtypeStruct((NFB, LQ, H * L * P), I32)
    return pl.pallas_call(
        _idxw_body,
        grid=(NFB,),
        in_specs=[spec] * 5,
        out_specs=[spec] * 8,
        out_shape=[shpi] * 4 + [shp] * 4,
    )(ox, oy, rx, ry, aw)


def _final_body(s0_ref, s1_ref, wo_ref, bo_ref, g0_ref, g1_ref, bg_ref, o_ref):
    t0 = jnp.dot(s0_ref[0], wo_ref[...], preferred_element_type=F32) + bo_ref[...]
    t1 = jnp.dot(s1_ref[0], wo_ref[...], preferred_element_type=F32) + bo_ref[...]
    o = (jnp.dot(t0, g0_ref[...], preferred_element_type=F32)
         + jnp.dot(t1, g1_ref[...], preferred_element_type=F32) + bg_ref[...])
    o_ref[...] = o[None]


def _final_combine(s0, s1, wot, bo2, g0t, g1t, bg2):
    mspec = lambda shape: pl.BlockSpec(shape, lambda i: (0, 0))
    return pl.pallas_call(
        _final_body,
        grid=(NB,),
        in_specs=[
            pl.BlockSpec((1, LQ, D), lambda i: (i, 0, 0)),
            pl.BlockSpec((1, LQ, D), lambda i: (i, 0, 0)),
            mspec((D, D)), mspec((1, D)), mspec((D, D)), mspec((D, D)),
            mspec((1, D)),
        ],
        out_specs=pl.BlockSpec((1, LQ, D), lambda i: (i, 0, 0)),
        out_shape=jax.ShapeDtypeStruct((NB, LQ, D), F32),
    )(s0, s1, wot, bo2, g0t, g1t, bg2)


def _make_sc_combine():
    mesh = plsc.VectorSubcoreMesh(core_axis_name="c", subcore_axis_name="s")

    @functools.partial(
        pl.kernel,
        mesh=mesh,
        out_type=jax.ShapeDtypeStruct((ROWS, HD), F32),
        scratch_types=[
            pltpu.VMEM((8, 128), I32),
            pltpu.VMEM((8, 128), F32),
            pltpu.VMEM((IDX_PER_CHUNK, HD), F32),
            pltpu.VMEM((8, 128), I32),
            pltpu.VMEM((8, 128), F32),
            pltpu.VMEM((IDX_PER_CHUNK, HD), F32),
            pltpu.VMEM((CHUNK, HD), F32),
            pltpu.SemaphoreType.DMA,
            pltpu.SemaphoreType.DMA,
        ],
        compiler_params=pltpu.CompilerParams(use_tc_tiling_on_sc=False),
    )
    def sc_combine(table, idxs, ws, out,
                   idx_v0, w_v0, rows_v0, idx_v1, w_v1, rows_v1,
                   out_v, sem0, sem1):
        wid = lax.axis_index("s") * 2 + lax.axis_index("c")
        bufs = ((idx_v0, w_v0, rows_v0, sem0), (idx_v1, w_v1, rows_v1, sem1))

        def load_fire(c, b):
            idx_v, w_v, rows_v, sem = bufs[b]
            row0 = wid * 128 + c * 2
            for cr in range(4):
                pltpu.sync_copy(idxs.at[cr, pl.ds(row0, 2)],
                                idx_v.at[pl.ds(cr * 2, 2)])
                pltpu.sync_copy(ws.at[cr, pl.ds(row0, 2)],
                                w_v.at[pl.ds(cr * 2, 2)])
            for j in range(8):
                pltpu.async_copy(table.at[idx_v.at[j]],
                                 rows_v.at[pl.ds(j * 128, 128)], sem)

        def drain_compute_store(c, b):
            idx_v, w_v, rows_v, sem = bufs[b]
            for j in range(8):
                pltpu.make_async_copy(table.at[idx_v.at[j]],
                                      rows_v.at[pl.ds(j * 128, 128)],
                                      sem).wait()

            def row_body(r, carry2):
                qq = r // 8
                h = r % 8
                a0 = jnp.zeros((16,), F32)
                a1 = jnp.zeros((16,), F32)
                for cr in range(4):
                    eb = (cr * 2 + qq) * 128 + h * 16
                    wvec = w_v[cr * 2 + qq, pl.ds(h * 16, 16)]
                    for j in range(16):
                        kk = eb + j
                        ws_j = wvec[j]
                        a0 = a0 + ws_j * rows_v[kk, pl.ds(0, 16)]
                        a1 = a1 + ws_j * rows_v[kk, pl.ds(16, 16)]
                out_v[r, pl.ds(0, 16)] = a0
                out_v[r, pl.ds(16, 16)] = a1
                return carry2

            lax.fori_loop(0, CHUNK, row_body, 0)
            pltpu.sync_copy(out_v, out.at[pl.ds(wid * RPW + c * CHUNK, CHUNK)])

        load_fire(0, 0)

        def pair_body(c2, carry):
            c = c2 * 2
            load_fire(c + 1, 1)
            drain_compute_store(c, 0)
            load_fire(c + 2, 0)
            drain_compute_store(c + 1, 1)
            return carry

        lax.fori_loop(0, NCHUNK // 2 - 1, pair_body, 0)
        c_last = NCHUNK - 2
        load_fire(c_last + 1, 1)
        drain_compute_store(c_last, 0)
        drain_compute_store(c_last + 1, 1)

    return sc_combine


_sc_cache = []


def _sc_combine(table, idxs, ws):
    if not _sc_cache:
        _sc_cache.append(_make_sc_combine())
    return _sc_cache[0](table, idxs, ws)


@jax.jit
def kernel(query_0, query_1, reference_points, input_flatten_0, input_flatten_1,
           input_spatial_shapes, input_level_start_index, W_value, b_value,
           W_off, b_off, W_attn, b_attn, W_out, b_out, W_agg, b_agg):
    flat = jnp.stack([input_flatten_0, input_flatten_1]).reshape(NFB, LEN_IN, D)
    vals = _matmul_bias(flat, W_value.T, b_value[None], 512)
    table = vals.reshape(TABLE_ROWS, HD)

    q = jnp.stack([query_0, query_1]).reshape(NFB, LQ, D)
    wcat = jnp.concatenate([W_off.T, W_attn.T], axis=1)
    bcat = jnp.concatenate([b_off, b_attn])[None]
    logits = _matmul_bias(q, wcat, bcat, LQ)

    off = logits[:, :, :H * L * P * 2].reshape(NFB, LQ, H * L * P, 2)
    ox = off[..., 0]
    oy = off[..., 1]

    al = logits[:, :, H * L * P * 2:].reshape(NF, NB, LQ, H, L * P)
    acat = al.transpose(1, 2, 3, 0, 4).reshape(NB, LQ, H, NF * L * P)
    asm = _softmax32(acat)
    aw = (asm.reshape(NB, LQ, H, NF, L * P).transpose(3, 0, 1, 2, 4)
          .reshape(NFB, LQ, H * L * P))

    rx = jnp.broadcast_to(reference_points[:, :, None, :, None, 0],
                          (NB, LQ, H, L, P)).reshape(NB, LQ, H * L * P)
    ry = jnp.broadcast_to(reference_points[:, :, None, :, None, 1],
                          (NB, LQ, H, L, P)).reshape(NB, LQ, H * L * P)
    rx = jnp.broadcast_to(rx[None], (NF, NB, LQ, H * L * P)).reshape(NFB, LQ, -1)
    ry = jnp.broadcast_to(ry[None], (NF, NB, LQ, H * L * P)).reshape(NFB, LQ, -1)

    i0, i1, i2, i3, w0, w1, w2, w3 = _idx_weights(ox, oy, rx, ry, aw)

    idx = jnp.stack([i0, i1, i2, i3]).reshape(4, NFB * LQ, H * L * P)
    wgt = jnp.stack([w0, w1, w2, w3]).reshape(4, NFB * LQ, H * L * P)

    sampled = _sc_combine(table, idx, wgt)
    s = sampled.reshape(NF, NB, LQ, D)

    return _final_combine(
        s[0], s[1], W_out.T, b_out[None],
        W_agg[:, :D].T, W_agg[:, D:].T, b_agg[None])
